# Initial kernel scaffold; baseline (speedup 1.0000x reference)
import jax
import jax.numpy as jnp
from jax.experimental import pallas as pl


def kernel(x, levels, borders):
    raise NotImplementedError("write your pallas kernel here")


# speedup vs baseline: 320.3628x; 320.3628x over previous
import functools

import jax
import jax.numpy as jnp
from jax import lax
from jax.experimental import pallas as pl
from jax.experimental.pallas import tpu as pltpu
from jax.experimental.pallas import tpu_sc as plsc

GROUP = 128
NLEV = 16
LANES = 16
VECS = GROUP // LANES


@functools.cache
def _make_sc_quantize(n_groups: int):
  info = plsc.get_sparse_core_info()
  nw = info.num_cores * info.num_subcores
  rows_per_w = n_groups // nw
  ch = 64
  n_chunks = rows_per_w // ch
  mesh = plsc.VectorSubcoreMesh(core_axis_name="c", subcore_axis_name="s")

  @functools.partial(
      pl.kernel,
      out_type=jax.ShapeDtypeStruct((n_groups, GROUP), jnp.float32),
      mesh=mesh,
      scratch_types=[
          pltpu.VMEM((ch, GROUP), jnp.float32),
          pltpu.VMEM((ch, NLEV), jnp.float32),
          pltpu.VMEM((ch, NLEV), jnp.float32),
          pltpu.VMEM((ch, GROUP), jnp.float32),
      ],
  )
  def body(x_hbm, lv_hbm, bd_hbm, out_hbm, x_v, l_v, b_v, o_v):
    wid = lax.axis_index("s") * info.num_cores + lax.axis_index("c")
    base = wid * rows_per_w

    def do_chunk(ci, carry):
      row0 = base + ci * ch
      pltpu.sync_copy(x_hbm.at[pl.ds(row0, ch)], x_v)
      pltpu.sync_copy(lv_hbm.at[pl.ds(row0, ch)], l_v)
      pltpu.sync_copy(bd_hbm.at[pl.ds(row0, ch)], b_v)

      def do_group(g, carry2):
        bvec = b_v[g, :]
        lvec = l_v[g, :]
        for v in range(VECS):
          xv = x_v[g, pl.ds(v * LANES, LANES)]
          idx = jnp.zeros((LANES,), jnp.int32)
          for w, off in ((8, 7), (4, 3), (2, 1), (1, 0)):
            probe = jnp.take_along_axis(
                bvec, idx + off, axis=0, mode="promise_in_bounds")
            idx = jnp.where(xv > probe, idx + w, idx)
          xq = jnp.take_along_axis(lvec, idx, axis=0, mode="promise_in_bounds")
          o_v[g, pl.ds(v * LANES, LANES)] = (xq - xv) + xv
        return carry2

      lax.fori_loop(0, ch, do_group, 0)
      pltpu.sync_copy(o_v, out_hbm.at[pl.ds(row0, ch)])
      return carry

    lax.fori_loop(0, n_chunks, do_chunk, 0)

  return body


def kernel(x, levels, borders):
  xg = x.reshape(-1, GROUP)
  bd = jnp.concatenate([borders, borders[:, -1:]], axis=1)
  out = _make_sc_quantize(xg.shape[0])(xg, levels, bd)
  return out.reshape(x.shape)
